# baseline (speedup 1.0000x reference)
import jax
import jax.numpy as jnp
from jax.experimental import pallas as pl
from jax.experimental.pallas import tpu as pltpu

B = 16384
N_HIDDEN = 256
N_MOTIF_PER_CLASS = 8
N_CLASS = 128
N_MOTIF = N_MOTIF_PER_CLASS * N_CLASS
TEMPERATURE = 0.2
EPSILON = 1e-4

BB = 2048
NBLK = B // BB


def _loss_kernel(z_ref, mt_ref, y_ref, out_ref, e_ref, mtb_ref, m2pe_ref):
    i = pl.program_id(0)

    @pl.when(i == 0)
    def _():
        ji = jax.lax.broadcasted_iota(jnp.int32, (N_MOTIF, N_CLASS), 0)
        ci = jax.lax.broadcasted_iota(jnp.int32, (N_MOTIF, N_CLASS), 1)
        e_ref[...] = ((ji // N_MOTIF_PER_CLASS) == ci).astype(jnp.bfloat16)
        mt = mt_ref[...]
        mtb_ref[...] = (mt * (-2.0)).astype(jnp.bfloat16)
        m2 = jnp.sum(mt * mt, axis=0, keepdims=True)
        m2pe_ref[...] = m2 + EPSILON

    z = z_ref[...]
    y = y_ref[...]

    xp2 = jax.lax.dot_general(
        z.astype(jnp.bfloat16), mtb_ref[...],
        dimension_numbers=(((1,), (0,)), ((), ())),
        preferred_element_type=jnp.float32,
    )
    z2 = jnp.sum(z * z, axis=1, keepdims=True)

    t = xp2 + z2
    den = t + m2pe_ref[...]
    u = (1.0 - EPSILON) * pl.reciprocal(den, approx=True, full_range=False)
    w = (u * (10.0 * u + 5.0)).astype(jnp.bfloat16)

    w_cls = jax.lax.dot_general(
        w, e_ref[...],
        dimension_numbers=(((1,), (0,)), ((), ())),
        preferred_element_type=jnp.float32,
    )

    cls_iota = jax.lax.broadcasted_iota(jnp.int32, (BB, N_CLASS), 1)
    mask = cls_iota == y

    total = jnp.sum(w_cls, axis=1, keepdims=True) + float(N_MOTIF)
    pos = (jnp.sum(jnp.where(mask, w_cls, 0.0), axis=1, keepdims=True)
           + float(N_MOTIF_PER_CLASS))

    partial = jnp.sum(jnp.log(pos / total)).reshape(1, 1)

    @pl.when(i == 0)
    def _():
        out_ref[...] = jnp.zeros((1, 1), jnp.float32)

    out_ref[...] += partial

    @pl.when(i == NBLK - 1)
    def _():
        out_ref[...] = out_ref[...] * (-1.0 / B)


@jax.jit
def kernel(z, y, motif_vector):
    mt = motif_vector.T
    y2 = y.reshape(B, 1)
    out = pl.pallas_call(
        _loss_kernel,
        grid=(NBLK,),
        in_specs=[
            pl.BlockSpec((BB, N_HIDDEN), lambda i: (i, 0)),
            pl.BlockSpec((N_HIDDEN, N_MOTIF), lambda i: (0, 0)),
            pl.BlockSpec((BB, 1), lambda i: (i, 0)),
        ],
        out_specs=pl.BlockSpec((1, 1), lambda i: (0, 0)),
        out_shape=jax.ShapeDtypeStruct((1, 1), jnp.float32),
        scratch_shapes=[
            pltpu.VMEM((N_MOTIF, N_CLASS), jnp.bfloat16),
            pltpu.VMEM((N_HIDDEN, N_MOTIF), jnp.bfloat16),
            pltpu.VMEM((1, N_MOTIF), jnp.float32),
        ],
    )(z, mt, y2)
    return out[0, 0]


# speedup vs baseline: 1.0329x; 1.0329x over previous
import jax
import jax.numpy as jnp
from jax.experimental import pallas as pl
from jax.experimental.pallas import tpu as pltpu

B = 16384
N_HIDDEN = 256
N_MOTIF_PER_CLASS = 8
N_CLASS = 128
N_MOTIF = N_MOTIF_PER_CLASS * N_CLASS
TEMPERATURE = 0.2
EPSILON = 1e-4

BB = 2048
NBLK = B // BB


def _loss_kernel(z_ref, mt_ref, y_ref, out_ref, e_ref, mtb_ref, m2pe_ref):
    i = pl.program_id(0)

    @pl.when(i == 0)
    def _():
        ji = jax.lax.broadcasted_iota(jnp.int32, (N_MOTIF, N_CLASS), 0)
        ci = jax.lax.broadcasted_iota(jnp.int32, (N_MOTIF, N_CLASS), 1)
        e_ref[...] = ((ji // N_MOTIF_PER_CLASS) == ci).astype(jnp.bfloat16)
        mt = mt_ref[...]
        mtb_ref[...] = (mt * (-2.0)).astype(jnp.bfloat16)
        m2 = jnp.sum(mt * mt, axis=0, keepdims=True)
        m2pe_ref[...] = m2 + EPSILON

    z = z_ref[...]
    y = y_ref[...]

    xp2 = jax.lax.dot_general(
        z.astype(jnp.bfloat16), mtb_ref[...],
        dimension_numbers=(((1,), (0,)), ((), ())),
        preferred_element_type=jnp.float32,
    )
    z2 = jnp.sum(z * z, axis=1, keepdims=True)

    t = xp2 + z2
    den = (t + m2pe_ref[...]).astype(jnp.bfloat16)
    u = jnp.bfloat16(1.0 - EPSILON) / den
    w = u * (jnp.bfloat16(10.0) * u + jnp.bfloat16(5.0))

    w_cls = jax.lax.dot_general(
        w, e_ref[...],
        dimension_numbers=(((1,), (0,)), ((), ())),
        preferred_element_type=jnp.float32,
    )

    cls_iota = jax.lax.broadcasted_iota(jnp.int32, (BB, N_CLASS), 1)
    mask = cls_iota == y

    total = jnp.sum(w_cls, axis=1, keepdims=True) + float(N_MOTIF)
    pos = (jnp.sum(jnp.where(mask, w_cls, 0.0), axis=1, keepdims=True)
           + float(N_MOTIF_PER_CLASS))

    partial = jnp.sum(jnp.log(pos / total)).reshape(1, 1)

    @pl.when(i == 0)
    def _():
        out_ref[...] = jnp.zeros((1, 1), jnp.float32)

    out_ref[...] += partial

    @pl.when(i == NBLK - 1)
    def _():
        out_ref[...] = out_ref[...] * (-1.0 / B)


@jax.jit
def kernel(z, y, motif_vector):
    mt = motif_vector.T
    y2 = y.reshape(B, 1)
    out = pl.pallas_call(
        _loss_kernel,
        grid=(NBLK,),
        in_specs=[
            pl.BlockSpec((BB, N_HIDDEN), lambda i: (i, 0)),
            pl.BlockSpec((N_HIDDEN, N_MOTIF), lambda i: (0, 0)),
            pl.BlockSpec((BB, 1), lambda i: (i, 0)),
        ],
        out_specs=pl.BlockSpec((1, 1), lambda i: (0, 0)),
        out_shape=jax.ShapeDtypeStruct((1, 1), jnp.float32),
        scratch_shapes=[
            pltpu.VMEM((N_MOTIF, N_CLASS), jnp.bfloat16),
            pltpu.VMEM((N_HIDDEN, N_MOTIF), jnp.bfloat16),
            pltpu.VMEM((1, N_MOTIF), jnp.float32),
        ],
    )(z, mt, y2)
    return out[0, 0]
